# baseline (speedup 1.0000x reference)
import functools

import jax
import jax.numpy as jnp
from jax import lax
from jax.experimental import pallas as pl
from jax.experimental.pallas import tpu as pltpu
from jax.experimental.pallas import tpu_sc as plsc

EMBED = 64
BATCH = 16384
LANES = 16
NUM_CORES = 2
NUM_SUBCORES = 16
NW = NUM_CORES * NUM_SUBCORES
CHUNK = 128
NCH = BATCH // (NW * CHUNK)

_MESH = plsc.VectorSubcoreMesh(
    core_axis_name="c", subcore_axis_name="s",
    num_cores=NUM_CORES, num_subcores=NUM_SUBCORES)


@functools.partial(
    pl.kernel,
    out_type=jax.ShapeDtypeStruct((NW, NCH, CHUNK, EMBED), jnp.float32),
    mesh=_MESH,
    scratch_types=[
        pltpu.VMEM((NCH, CHUNK), jnp.int32),
        pltpu.VMEM((NCH, CHUNK, EMBED), jnp.float32),
        pltpu.VMEM((NCH, CHUNK, EMBED), jnp.float32),
        pltpu.SemaphoreType.DMA,
        pltpu.SemaphoreType.DMA,
    ],
    compiler_params=pltpu.CompilerParams(use_tc_tiling_on_sc=False),
)
def _sc_embed(x_hbm, idx_hbm, table_hbm, out_hbm,
              idx_v, x_v, rows_v, gsem, xsem):
    wid = lax.axis_index("s") * NUM_CORES + lax.axis_index("c")

    pltpu.sync_copy(idx_hbm.at[wid], idx_v)
    gathers = [
        pltpu.async_copy(table_hbm.at[idx_v.at[ch]], rows_v.at[ch], gsem)
        for ch in range(NCH)
    ]
    x_cp = pltpu.async_copy(x_hbm.at[wid], x_v, xsem)
    for cp in gathers:
        cp.wait()
    x_cp.wait()

    def body(r, carry):
        for ch in range(NCH):
            for d in range(EMBED // LANES):
                sl = pl.ds(d * LANES, LANES)
                rows_v[ch, r, sl] = rows_v[ch, r, sl] * x_v[ch, r, sl]
        return carry

    lax.fori_loop(0, CHUNK, body, 0)

    pltpu.sync_copy(rows_v, out_hbm.at[wid])


def kernel(x, id, table):
    idx = id.astype(jnp.int32).reshape(NW, NCH, CHUNK)
    x_r = x.reshape(NW, NCH, CHUNK, EMBED)
    out = _sc_embed(x_r, idx, table)
    return out.reshape(BATCH, EMBED)


# speedup vs baseline: 1.6821x; 1.6821x over previous
import functools

import jax
import jax.numpy as jnp
from jax import lax
from jax.experimental import pallas as pl
from jax.experimental.pallas import tpu as pltpu
from jax.experimental.pallas import tpu_sc as plsc

EMBED = 64
BATCH = 16384
LANES = 16
NUM_CORES = 2
NUM_SUBCORES = 16
NW = NUM_CORES * NUM_SUBCORES
CHUNK = 128
NCH = BATCH // (NW * CHUNK)

_MESH = plsc.VectorSubcoreMesh(
    core_axis_name="c", subcore_axis_name="s",
    num_cores=NUM_CORES, num_subcores=NUM_SUBCORES)


@functools.partial(
    pl.kernel,
    out_type=jax.ShapeDtypeStruct((NW, NCH, CHUNK, EMBED), jnp.float32),
    mesh=_MESH,
    scratch_types=[
        pltpu.VMEM((NCH, CHUNK), jnp.int32),
        pltpu.VMEM((CHUNK, EMBED), jnp.float32),
        pltpu.VMEM((CHUNK, EMBED), jnp.float32),
        pltpu.SemaphoreType.DMA,
        pltpu.SemaphoreType.DMA,
    ],
)
def _sc_embed(x_hbm, idx_hbm, table_hbm, out_hbm,
              idx_v, x_v, rows_v, gsem, xsem):
    wid = lax.axis_index("s") * NUM_CORES + lax.axis_index("c")

    pltpu.sync_copy(idx_hbm.at[wid], idx_v)

    for c in range(NCH):
        x_cp = pltpu.async_copy(x_hbm.at[wid].at[c], x_v, xsem)

        def issue(g, carry):
            vec = idx_v[c, pl.ds(g * LANES, LANES)]
            for l in range(LANES):
                pltpu.async_copy(
                    table_hbm.at[vec[l]], rows_v.at[g * LANES + l], gsem)
            return carry

        lax.fori_loop(0, CHUNK // LANES, issue, 0)
        pltpu.make_async_copy(
            table_hbm.at[pl.ds(0, CHUNK)], rows_v, gsem).wait()
        x_cp.wait()

        def mul(r, carry):
            for d in range(EMBED // LANES):
                sl = pl.ds(d * LANES, LANES)
                rows_v[r, sl] = rows_v[r, sl] * x_v[r, sl]
            return carry

        lax.fori_loop(0, CHUNK, mul, 0)

        pltpu.sync_copy(rows_v, out_hbm.at[wid].at[c])


def kernel(x, id, table):
    idx = id.astype(jnp.int32).reshape(NW, NCH, CHUNK)
    x_r = x.reshape(NW, NCH, CHUNK, EMBED)
    out = _sc_embed(x_r, idx, table)
    return out.reshape(BATCH, EMBED)
